# baseline (speedup 1.0000x reference)
import functools
import math

import jax
import jax.numpy as jnp
from jax import lax
from jax.experimental import pallas as pl
from jax.experimental.pallas import tpu as pltpu
from jax.experimental.pallas import tpu_sc as plsc

V = 1000000
F = 26
D = 64
H = 64
B = 16384
PL = F // 2
NC, NS = 2, 16
NW = NC * NS
RW = B // NW
R = 4
CH = R * F
NCHUNK = RW // R
NBUF = 4
NSB = 2
GR = 2 * R
MFG = 32
LANES = 16
BN_INV = 1.0 / math.sqrt(1.0 + 1e-3)

VB = 4096


def _prep_body(mf_t_ref, mlp_t_ref, out_ref):
    out_ref[:, :D] = mf_t_ref[...].T
    out_ref[:, D:] = mlp_t_ref[...].T


_prep = pl.pallas_call(
    _prep_body,
    grid=(pl.cdiv(V, VB),),
    in_specs=[
        pl.BlockSpec((D, VB), lambda i: (0, i)),
        pl.BlockSpec((D, VB), lambda i: (0, i)),
    ],
    out_specs=pl.BlockSpec((VB, 2 * D), lambda i: (i, 0)),
    out_shape=jax.ShapeDtypeStruct((V, 2 * D), jnp.float32),
)


def _sc_body(idx_hbm, c_hbm, mfprod_hbm, planes_hbm, idx_v, mfres, *rest):
    bufs = rest[0:NBUF]
    stages = rest[NBUF:NBUF + NSB]
    gsem = rest[NBUF + NSB:2 * NBUF + NSB]
    ssem = rest[2 * NBUF + NSB:2 * NBUF + 2 * NSB]

    cid = lax.axis_index("c")
    sid = lax.axis_index("s")
    wid = sid * NC + cid
    row0 = wid * RW

    pltpu.sync_copy(idx_hbm.at[pl.ds(wid * RW * F, RW * F)], idx_v)

    def start_gather(j, b):
        pltpu.async_copy(c_hbm.at[idx_v.at[pl.ds(j * CH, CH)]], bufs[b],
                         gsem[b])

    def drain_group(g, sb):
        pltpu.make_async_copy(
            stages[sb], planes_hbm.at[wid * (NCHUNK // 2) + g],
            ssem[sb]).wait()

    for b in range(NBUF):
        start_gather(b, b)

    @pl.loop(0, NCHUNK // 2, step=NSB)
    def _(gbase):
        for gb in range(NSB):
            g = gbase + gb
            @pl.when(g >= NSB)
            def _():
                drain_group(g - NSB, gb)

            for half in range(2):
                j = g * 2 + half
                b = gb * 2 + half
                pltpu.make_async_copy(
                    c_hbm.at[idx_v.at[pl.ds(j * CH, CH)]], bufs[b],
                    gsem[b]).wait()
                for r in range(R):
                    for k in range(D // LANES):
                        vals = [bufs[b][r * F + f, pl.ds(k * LANES, LANES)]
                                for f in range(F)]
                        while len(vals) > 1:
                            nxt = [a * c2 for a, c2 in
                                   zip(vals[0::2], vals[1::2])]
                            if len(vals) % 2:
                                nxt.append(vals[-1])
                            vals = nxt
                        mfres[(j % MFG) * R + r,
                              pl.ds(k * LANES, LANES)] = vals[0]
                    row = half * R + r
                    for c in range(PL):
                        for k in range(D // LANES):
                            stages[gb][c, row, pl.ds(k * LANES, LANES)] = (
                                bufs[b][r * F + 2 * c,
                                        pl.ds(D + k * LANES, LANES)])
                            stages[gb][c, row,
                                       pl.ds(D + k * LANES, LANES)] = (
                                bufs[b][r * F + 2 * c + 1,
                                        pl.ds(D + k * LANES, LANES)])

                @pl.when(j % MFG == MFG - 1)
                def _():
                    pltpu.sync_copy(
                        mfres,
                        mfprod_hbm.at[pl.ds(
                            pl.multiple_of(row0 + (j - (MFG - 1)) * R,
                                           MFG * R),
                            MFG * R)])

                @pl.when(j + NBUF < NCHUNK)
                def _():
                    start_gather(j + NBUF, b)

            pltpu.async_copy(stages[gb],
                             planes_hbm.at[wid * (NCHUNK // 2) + g],
                             ssem[gb])

    for gb in range(NSB):
        drain_group(NCHUNK // 2 - NSB + gb, gb)


_sc_gather = pl.kernel(
    _sc_body,
    out_type=[
        jax.ShapeDtypeStruct((B, D), jnp.float32),
        jax.ShapeDtypeStruct((B // GR, PL, GR, 2 * D),
                             jnp.float32),
    ],
    mesh=plsc.VectorSubcoreMesh(core_axis_name="c", subcore_axis_name="s",
                                num_cores=NC, num_subcores=NS),
    compiler_params=pltpu.CompilerParams(use_tc_tiling_on_sc=True),
    scratch_types=(
        [pltpu.VMEM((RW * F,), jnp.int32),
         pltpu.VMEM((MFG * R, D), jnp.float32)]
        + [pltpu.VMEM((CH, 2 * D), jnp.float32)] * NBUF
        + [pltpu.VMEM((PL, GR, 2 * D), jnp.float32)] * NSB
        + [pltpu.SemaphoreType.DMA] * (NBUF + NSB)
    ),
)


BT = 2048


def _tc_body(planes_ref, mf_ref, w1_ref, b1_ref, gamma_ref, beta_ref,
             wfm_ref, wfh_ref, bf_ref, out_ref):
    h = None
    for c in range(PL):
        xc = planes_ref[:, c].reshape(BT, 2 * D)
        d = jnp.dot(xc, w1_ref[c], preferred_element_type=jnp.float32)
        h = d if h is None else h + d
    h = jnp.maximum(h + b1_ref[...], 0.0)
    h = h * (gamma_ref[...] * BN_INV) + beta_ref[...]
    out_ref[...] = (
        jnp.dot(mf_ref[...], wfm_ref[...], preferred_element_type=jnp.float32)
        + jnp.dot(h, wfh_ref[...], preferred_element_type=jnp.float32)
        + bf_ref[0, 0]
    )


def _tc_dense(planes, mfprod, W1r, b1, gamma, beta, Wf, bf):
    return pl.pallas_call(
        _tc_body,
        grid=(B // BT,),
        in_specs=[
            pl.BlockSpec((BT // GR, PL, GR, 2 * D), lambda i: (i, 0, 0, 0)),
            pl.BlockSpec((BT, D), lambda i: (i, 0)),
            pl.BlockSpec((PL, 2 * D, H), lambda i: (0, 0, 0)),
            pl.BlockSpec((1, H), lambda i: (0, 0)),
            pl.BlockSpec((1, H), lambda i: (0, 0)),
            pl.BlockSpec((1, H), lambda i: (0, 0)),
            pl.BlockSpec((D, 1), lambda i: (0, 0)),
            pl.BlockSpec((H, 1), lambda i: (0, 0)),
            pl.BlockSpec((1, 1), lambda i: (0, 0)),
        ],
        out_specs=pl.BlockSpec((BT, 1), lambda i: (i, 0)),
        out_shape=jax.ShapeDtypeStruct((B, 1), jnp.float32),
    )(planes, mfprod,
      W1r, b1.reshape(1, H), gamma.reshape(1, H), beta.reshape(1, H),
      Wf[:D], Wf[D:], bf.reshape(1, 1))


def kernel(one_hot_features, mf_table, mlp_table, W1, b1, gamma, beta,
           Wf, bf):
    idx = one_hot_features.astype(jnp.int32).reshape(B * F)
    combined = _prep(mf_table.T, mlp_table.T)
    mfprod, planes = _sc_gather(idx, combined)
    W1r = W1.reshape(PL, 2 * D, H)
    return _tc_dense(planes, mfprod, W1r, b1, gamma, beta, Wf, bf)


# speedup vs baseline: 1.1363x; 1.1363x over previous
import functools
import math

import jax
import jax.numpy as jnp
from jax import lax
from jax.experimental import pallas as pl
from jax.experimental.pallas import tpu as pltpu
from jax.experimental.pallas import tpu_sc as plsc

V = 1000000
F = 26
D = 64
H = 64
B = 16384
PL = F // 2
NC, NS = 2, 16
NW = NC * NS
RW = B // NW
R = 4
CH = R * F
NCHUNK = RW // R
NBUF = 4
NSB = 2
GR = 2 * R
MFG = 32
LANES = 16
BN_INV = 1.0 / math.sqrt(1.0 + 1e-3)

VB = 16384


def _prep_body(mf_t_ref, mlp_t_ref, out_ref):
    out_ref[:, :D] = mf_t_ref[...].T
    out_ref[:, D:] = mlp_t_ref[...].T


_prep = pl.pallas_call(
    _prep_body,
    grid=(pl.cdiv(V, VB),),
    in_specs=[
        pl.BlockSpec((D, VB), lambda i: (0, i)),
        pl.BlockSpec((D, VB), lambda i: (0, i)),
    ],
    out_specs=pl.BlockSpec((VB, 2 * D), lambda i: (i, 0)),
    out_shape=jax.ShapeDtypeStruct((V, 2 * D), jnp.float32),
)


def _sc_body(idx_hbm, c_hbm, mfprod_hbm, planes_hbm, idx_v, mfres, *rest):
    bufs = rest[0:NBUF]
    stages = rest[NBUF:NBUF + NSB]
    gsem = rest[NBUF + NSB:2 * NBUF + NSB]
    ssem = rest[2 * NBUF + NSB:2 * NBUF + 2 * NSB]

    cid = lax.axis_index("c")
    sid = lax.axis_index("s")
    wid = sid * NC + cid
    row0 = wid * RW

    pltpu.sync_copy(idx_hbm.at[pl.ds(wid * RW * F, RW * F)], idx_v)

    def start_gather(j, b):
        pltpu.async_copy(c_hbm.at[idx_v.at[pl.ds(j * CH, CH)]], bufs[b],
                         gsem[b])

    def drain_group(g, sb):
        pltpu.make_async_copy(
            stages[sb], planes_hbm.at[wid * (NCHUNK // 2) + g],
            ssem[sb]).wait()

    for b in range(NBUF):
        start_gather(b, b)

    @pl.loop(0, NCHUNK // 2, step=NSB)
    def _(gbase):
        for gb in range(NSB):
            g = gbase + gb
            @pl.when(g >= NSB)
            def _():
                drain_group(g - NSB, gb)

            for half in range(2):
                j = g * 2 + half
                b = gb * 2 + half
                pltpu.make_async_copy(
                    c_hbm.at[idx_v.at[pl.ds(j * CH, CH)]], bufs[b],
                    gsem[b]).wait()
                for r in range(R):
                    for k in range(D // LANES):
                        vals = [bufs[b][r * F + f, pl.ds(k * LANES, LANES)]
                                for f in range(F)]
                        while len(vals) > 1:
                            nxt = [a * c2 for a, c2 in
                                   zip(vals[0::2], vals[1::2])]
                            if len(vals) % 2:
                                nxt.append(vals[-1])
                            vals = nxt
                        mfres[(j % MFG) * R + r,
                              pl.ds(k * LANES, LANES)] = vals[0]
                    row = half * R + r
                    for c in range(PL):
                        for k in range(D // LANES):
                            stages[gb][c, row, pl.ds(k * LANES, LANES)] = (
                                bufs[b][r * F + 2 * c,
                                        pl.ds(D + k * LANES, LANES)])
                            stages[gb][c, row,
                                       pl.ds(D + k * LANES, LANES)] = (
                                bufs[b][r * F + 2 * c + 1,
                                        pl.ds(D + k * LANES, LANES)])

                @pl.when(j % MFG == MFG - 1)
                def _():
                    pltpu.sync_copy(
                        mfres,
                        mfprod_hbm.at[pl.ds(
                            pl.multiple_of(row0 + (j - (MFG - 1)) * R,
                                           MFG * R),
                            MFG * R)])

                @pl.when(j + NBUF < NCHUNK)
                def _():
                    start_gather(j + NBUF, b)

            pltpu.async_copy(stages[gb],
                             planes_hbm.at[wid * (NCHUNK // 2) + g],
                             ssem[gb])

    for gb in range(NSB):
        drain_group(NCHUNK // 2 - NSB + gb, gb)


_sc_gather = pl.kernel(
    _sc_body,
    out_type=[
        jax.ShapeDtypeStruct((B, D), jnp.float32),
        jax.ShapeDtypeStruct((B // GR, PL, GR, 2 * D),
                             jnp.float32),
    ],
    mesh=plsc.VectorSubcoreMesh(core_axis_name="c", subcore_axis_name="s",
                                num_cores=NC, num_subcores=NS),
    compiler_params=pltpu.CompilerParams(use_tc_tiling_on_sc=True),
    scratch_types=(
        [pltpu.VMEM((RW * F,), jnp.int32),
         pltpu.VMEM((MFG * R, D), jnp.float32)]
        + [pltpu.VMEM((CH, 2 * D), jnp.float32)] * NBUF
        + [pltpu.VMEM((PL, GR, 2 * D), jnp.float32)] * NSB
        + [pltpu.SemaphoreType.DMA] * (NBUF + NSB)
    ),
)


BT = 2048


def _tc_body(planes_ref, mf_ref, w1_ref, b1_ref, gamma_ref, beta_ref,
             wfm_ref, wfh_ref, bf_ref, out_ref):
    h = None
    for c in range(PL):
        xc = planes_ref[:, c].reshape(BT, 2 * D)
        d = jnp.dot(xc, w1_ref[c], preferred_element_type=jnp.float32)
        h = d if h is None else h + d
    h = jnp.maximum(h + b1_ref[...], 0.0)
    h = h * (gamma_ref[...] * BN_INV) + beta_ref[...]
    out_ref[...] = (
        jnp.dot(mf_ref[...], wfm_ref[...], preferred_element_type=jnp.float32)
        + jnp.dot(h, wfh_ref[...], preferred_element_type=jnp.float32)
        + bf_ref[0, 0]
    )


def _tc_dense(planes, mfprod, W1r, b1, gamma, beta, Wf, bf):
    return pl.pallas_call(
        _tc_body,
        grid=(B // BT,),
        in_specs=[
            pl.BlockSpec((BT // GR, PL, GR, 2 * D), lambda i: (i, 0, 0, 0)),
            pl.BlockSpec((BT, D), lambda i: (i, 0)),
            pl.BlockSpec((PL, 2 * D, H), lambda i: (0, 0, 0)),
            pl.BlockSpec((1, H), lambda i: (0, 0)),
            pl.BlockSpec((1, H), lambda i: (0, 0)),
            pl.BlockSpec((1, H), lambda i: (0, 0)),
            pl.BlockSpec((D, 1), lambda i: (0, 0)),
            pl.BlockSpec((H, 1), lambda i: (0, 0)),
            pl.BlockSpec((1, 1), lambda i: (0, 0)),
        ],
        out_specs=pl.BlockSpec((BT, 1), lambda i: (i, 0)),
        out_shape=jax.ShapeDtypeStruct((B, 1), jnp.float32),
    )(planes, mfprod,
      W1r, b1.reshape(1, H), gamma.reshape(1, H), beta.reshape(1, H),
      Wf[:D], Wf[D:], bf.reshape(1, 1))


def kernel(one_hot_features, mf_table, mlp_table, W1, b1, gamma, beta,
           Wf, bf):
    idx = one_hot_features.astype(jnp.int32).reshape(B * F)
    combined = _prep(mf_table.T, mlp_table.T)
    mfprod, planes = _sc_gather(idx, combined)
    W1r = W1.reshape(PL, 2 * D, H)
    return _tc_dense(planes, mfprod, W1r, b1, gamma, beta, Wf, bf)


# speedup vs baseline: 1.1364x; 1.0001x over previous
import math

import jax
import jax.numpy as jnp
from jax import lax
from jax.experimental import pallas as pl
from jax.experimental.pallas import tpu as pltpu
from jax.experimental.pallas import tpu_sc as plsc

V = 1000000
F = 26
D = 64
H = 64
B = 16384
PL = F // 2
NC, NS = 2, 16
NW = NC * NS
RW = B // NW
R = 4
CH = R * F
NCHUNK = RW // R
NBUF = 4
NSB = 2
GR = 2 * R
MFG = 32
LANES = 16
BN_INV = 1.0 / math.sqrt(1.0 + 1e-3)

VB = 16384


def _prep_body(mf_t_ref, mlp_t_ref, out_ref):
    out_ref[:, :D] = mf_t_ref[...].T
    out_ref[:, D:] = mlp_t_ref[...].T


_prep = pl.pallas_call(
    _prep_body,
    grid=(pl.cdiv(V, VB),),
    in_specs=[
        pl.BlockSpec((D, VB), lambda i: (0, i)),
        pl.BlockSpec((D, VB), lambda i: (0, i)),
    ],
    out_specs=pl.BlockSpec((VB, 2 * D), lambda i: (i, 0)),
    out_shape=jax.ShapeDtypeStruct((V, 2 * D), jnp.float32),
)


def _sc_body(idx_hbm, c_hbm, mfprod_hbm, planes_hbm, idx_v, mfres, *rest):
    bufs = rest[0:NBUF]
    stages = rest[NBUF:NBUF + NSB]
    gsem = rest[NBUF + NSB:2 * NBUF + NSB]
    ssem = rest[2 * NBUF + NSB:2 * NBUF + 2 * NSB]

    cid = lax.axis_index("c")
    sid = lax.axis_index("s")
    wid = sid * NC + cid
    row0 = wid * RW

    pltpu.sync_copy(idx_hbm.at[pl.ds(wid * RW * F, RW * F)], idx_v)

    def start_gather(j, b):
        pltpu.async_copy(c_hbm.at[idx_v.at[pl.ds(j * CH, CH)]], bufs[b],
                         gsem[b])

    def drain_group(g, sb):
        pltpu.make_async_copy(
            stages[sb], planes_hbm.at[wid * (NCHUNK // 2) + g],
            ssem[sb]).wait()

    for b in range(NBUF):
        start_gather(b, b)

    @pl.loop(0, NCHUNK // 2, step=NSB)
    def _(gbase):
        for gb in range(NSB):
            g = gbase + gb
            @pl.when(g >= NSB)
            def _():
                drain_group(g - NSB, gb)

            for half in range(2):
                j = g * 2 + half
                b = gb * 2 + half
                pltpu.make_async_copy(
                    c_hbm.at[idx_v.at[pl.ds(j * CH, CH)]], bufs[b],
                    gsem[b]).wait()
                for r in range(R):
                    for k in range(D // LANES):
                        vals = [bufs[b][r * F + f, pl.ds(k * LANES, LANES)]
                                for f in range(F)]
                        while len(vals) > 1:
                            nxt = [a * c2 for a, c2 in
                                   zip(vals[0::2], vals[1::2])]
                            if len(vals) % 2:
                                nxt.append(vals[-1])
                            vals = nxt
                        mfres[(j % MFG) * R + r,
                              pl.ds(k * LANES, LANES)] = vals[0]
                    row = half * R + r
                    for c in range(PL):
                        for k in range(D // LANES):
                            stages[gb][c, row, pl.ds(k * LANES, LANES)] = (
                                bufs[b][r * F + 2 * c,
                                        pl.ds(D + k * LANES, LANES)])
                            stages[gb][c, row,
                                       pl.ds(D + k * LANES, LANES)] = (
                                bufs[b][r * F + 2 * c + 1,
                                        pl.ds(D + k * LANES, LANES)])

                @pl.when(j % MFG == MFG - 1)
                def _():
                    pltpu.sync_copy(
                        mfres,
                        mfprod_hbm.at[pl.ds(
                            pl.multiple_of(row0 + (j - (MFG - 1)) * R,
                                           MFG * R),
                            MFG * R)])

                @pl.when(j + NBUF < NCHUNK)
                def _():
                    start_gather(j + NBUF, b)

            pltpu.async_copy(stages[gb],
                             planes_hbm.at[wid * (NCHUNK // 2) + g],
                             ssem[gb])

    for gb in range(NSB):
        drain_group(NCHUNK // 2 - NSB + gb, gb)


_sc_gather = pl.kernel(
    _sc_body,
    out_type=[
        jax.ShapeDtypeStruct((B, D), jnp.float32),
        jax.ShapeDtypeStruct((B // GR, PL, GR, 2 * D),
                             jnp.float32),
    ],
    mesh=plsc.VectorSubcoreMesh(core_axis_name="c", subcore_axis_name="s",
                                num_cores=NC, num_subcores=NS),
    compiler_params=pltpu.CompilerParams(use_tc_tiling_on_sc=True),
    scratch_types=(
        [pltpu.VMEM((RW * F,), jnp.int32),
         pltpu.VMEM((MFG * R, D), jnp.float32)]
        + [pltpu.VMEM((CH, 2 * D), jnp.float32)] * NBUF
        + [pltpu.VMEM((PL, GR, 2 * D), jnp.float32)] * NSB
        + [pltpu.SemaphoreType.DMA] * (NBUF + NSB)
    ),
)


BT = 2048


def _tc_body(planes_ref, mf_ref, w1_ref, b1_ref, gamma_ref, beta_ref,
             wfm_ref, wfh_ref, bf_ref, out_ref):
    h = None
    for c in range(PL):
        xc = planes_ref[:, c].reshape(BT, 2 * D)
        d = jnp.dot(xc, w1_ref[c], preferred_element_type=jnp.float32)
        h = d if h is None else h + d
    h = jnp.maximum(h + b1_ref[...], 0.0)
    h = h * (gamma_ref[...] * BN_INV) + beta_ref[...]
    out_ref[...] = (
        jnp.dot(mf_ref[...], wfm_ref[...], preferred_element_type=jnp.float32)
        + jnp.dot(h, wfh_ref[...], preferred_element_type=jnp.float32)
        + bf_ref[0, 0]
    )


def _tc_dense(planes, mfprod, W1r, b1, gamma, beta, Wf, bf):
    return pl.pallas_call(
        _tc_body,
        grid=(B // BT,),
        in_specs=[
            pl.BlockSpec((BT // GR, PL, GR, 2 * D), lambda i: (i, 0, 0, 0)),
            pl.BlockSpec((BT, D), lambda i: (i, 0)),
            pl.BlockSpec((PL, 2 * D, H), lambda i: (0, 0, 0)),
            pl.BlockSpec((1, H), lambda i: (0, 0)),
            pl.BlockSpec((1, H), lambda i: (0, 0)),
            pl.BlockSpec((1, H), lambda i: (0, 0)),
            pl.BlockSpec((D, 1), lambda i: (0, 0)),
            pl.BlockSpec((H, 1), lambda i: (0, 0)),
            pl.BlockSpec((1, 1), lambda i: (0, 0)),
        ],
        out_specs=pl.BlockSpec((BT, 1), lambda i: (i, 0)),
        out_shape=jax.ShapeDtypeStruct((B, 1), jnp.float32),
    )(planes, mfprod,
      W1r, b1.reshape(1, H), gamma.reshape(1, H), beta.reshape(1, H),
      Wf[:D], Wf[D:], bf.reshape(1, 1))


def kernel(one_hot_features, mf_table, mlp_table, W1, b1, gamma, beta,
           Wf, bf):
    idx = one_hot_features.astype(jnp.int32).reshape(B * F)
    combined = _prep(mf_table.T, mlp_table.T)
    mfprod, planes = _sc_gather(idx, combined)
    W1r = W1.reshape(PL, 2 * D, H)
    return _tc_dense(planes, mfprod, W1r, b1, gamma, beta, Wf, bf)
